# baseline (speedup 1.0000x reference)
import functools

import jax
import jax.numpy as jnp
from jax import lax
from jax.experimental import pallas as pl
from jax.experimental.pallas import tpu as pltpu
from jax.experimental.pallas import tpu_sc as plsc


@functools.lru_cache(maxsize=None)
def _build_gather(B, V, D):
    NC, NS = 2, 16
    NW = NC * NS
    b_per_w = B // NW
    CH = 64
    n_ch = b_per_w // CH
    NBUF = 4

    mesh = plsc.VectorSubcoreMesh(core_axis_name="c", subcore_axis_name="s")

    @functools.partial(
        pl.kernel,
        mesh=mesh,
        out_type=jax.ShapeDtypeStruct((B, D), jnp.float32),
        scratch_types=[
            pltpu.VMEM((b_per_w,), jnp.int32),
            pltpu.VMEM((NBUF, CH, D), jnp.float32),
        ]
        + [pltpu.SemaphoreType.DMA] * (2 * NBUF),
    )
    def k(mem, idx_hbm, rows_hbm, idx_v, rows_v, *sems):
        g = sems[0:NBUF]
        o = sems[NBUF:2 * NBUF]

        wid = lax.axis_index("s") * NC + lax.axis_index("c")
        base = wid * b_per_w
        pltpu.sync_copy(idx_hbm.at[pl.ds(base, b_per_w)], idx_v)

        handles = {}

        def issue_gather(c):
            buf = c % NBUF
            handles[("g", c)] = pltpu.async_copy(
                mem.at[idx_v.at[pl.ds(c * CH, CH)]], rows_v.at[buf], g[buf])

        for c in range(min(NBUF, n_ch)):
            issue_gather(c)
        for c in range(n_ch):
            buf = c % NBUF
            handles[("g", c)].wait()
            handles[("o", c)] = pltpu.async_copy(
                rows_v.at[buf], rows_hbm.at[pl.ds(base + c * CH, CH)], o[buf])
            if c + NBUF < n_ch:
                handles[("o", c)].wait()
                issue_gather(c + NBUF)
        for c in range(max(0, n_ch - NBUF), n_ch):
            handles[("o", c)].wait()

    return k


@functools.lru_cache(maxsize=None)
def _build_epilogue(B, D, half):
    BLK = 16384
    n = B // BLK
    NB = min(2, n)

    def body(rows_hbm, emb_ref, num_ref, buf, sem):
        i = pl.program_id(0)

        @pl.when(i == 0)
        def _():
            pltpu.make_async_copy(
                rows_hbm.at[pl.ds(0, BLK)], buf.at[0], sem.at[0]).start()

        @pl.when(i + 1 < n)
        def _():
            pltpu.make_async_copy(
                rows_hbm.at[pl.ds((i + 1) * BLK, BLK)],
                buf.at[(i + 1) % 2], sem.at[(i + 1) % 2]).start()

        pltpu.make_async_copy(
            rows_hbm.at[pl.ds(i * BLK, BLK)],
            buf.at[i % 2], sem.at[i % 2]).wait()
        x = buf[i % 2]
        eye = (lax.broadcasted_iota(jnp.int32, (half, half), 0)
               == lax.broadcasted_iota(jnp.int32, (half, half), 1)
               ).astype(jnp.float32)
        dn = (((1,), (1,)), ((), ()))
        num_t = lax.dot_general(eye, x[:, :half], dn,
                                preferred_element_type=jnp.float32)
        den_t = lax.dot_general(eye, x[:, half:], dn,
                                preferred_element_type=jnp.float32)
        num_ref[...] = num_t
        emb_ref[...] = num_t / den_t

    return pl.pallas_call(
        body,
        grid=(n,),
        in_specs=[pl.BlockSpec(memory_space=pltpu.MemorySpace.HBM)],
        out_specs=[
            pl.BlockSpec((half, BLK), lambda i: (0, i)),
            pl.BlockSpec((half, BLK), lambda i: (0, i)),
        ],
        out_shape=(
            jax.ShapeDtypeStruct((half, B), jnp.float32),
            jax.ShapeDtypeStruct((half, B), jnp.float32),
        ),
        scratch_shapes=[
            pltpu.VMEM((NB, BLK, D), jnp.float32),
            pltpu.SemaphoreType.DMA((NB,)),
        ],
    )


def kernel(memory, nodes, memory_dim):
    V, D = memory.shape
    B = nodes.shape[0]
    half = D // 2
    rows = _build_gather(B, V, D)(memory, nodes.astype(jnp.int32))
    rows = pltpu.with_memory_space_constraint(rows, pltpu.MemorySpace.HBM)
    emb_t, num_t = _build_epilogue(B, D, half)(rows)
    return (emb_t.T, num_t.T)


# speedup vs baseline: 1.0561x; 1.0561x over previous
import functools

import jax
import jax.numpy as jnp
from jax import lax
from jax.experimental import pallas as pl
from jax.experimental.pallas import tpu as pltpu
from jax.experimental.pallas import tpu_sc as plsc


@functools.lru_cache(maxsize=None)
def _build_gather(B, V, D):
    NC, NS = 2, 16
    NW = NC * NS
    b_per_w = B // NW
    CH = 64
    n_ch = b_per_w // CH
    NBUF = 8

    mesh = plsc.VectorSubcoreMesh(core_axis_name="c", subcore_axis_name="s")

    @functools.partial(
        pl.kernel,
        mesh=mesh,
        out_type=jax.ShapeDtypeStruct((B, D), jnp.float32),
        scratch_types=[
            pltpu.VMEM((b_per_w,), jnp.int32),
            pltpu.VMEM((NBUF, CH, D), jnp.float32),
        ]
        + [pltpu.SemaphoreType.DMA] * (2 * NBUF),
    )
    def k(mem, idx_hbm, rows_hbm, idx_v, rows_v, *sems):
        g = sems[0:NBUF]
        o = sems[NBUF:2 * NBUF]

        wid = lax.axis_index("s") * NC + lax.axis_index("c")
        base = wid * b_per_w
        pltpu.sync_copy(idx_hbm.at[pl.ds(base, b_per_w)], idx_v)

        handles = {}

        def issue_gather(c):
            buf = c % NBUF
            handles[("g", c)] = pltpu.async_copy(
                mem.at[idx_v.at[pl.ds(c * CH, CH)]], rows_v.at[buf], g[buf])

        for c in range(min(NBUF, n_ch)):
            issue_gather(c)
        for c in range(n_ch):
            buf = c % NBUF
            handles[("g", c)].wait()
            handles[("o", c)] = pltpu.async_copy(
                rows_v.at[buf], rows_hbm.at[pl.ds(base + c * CH, CH)], o[buf])
            if c + NBUF < n_ch:
                handles[("o", c)].wait()
                issue_gather(c + NBUF)
        for c in range(max(0, n_ch - NBUF), n_ch):
            handles[("o", c)].wait()

    return k


@functools.lru_cache(maxsize=None)
def _build_epilogue(B, D, half):
    BLK = 8192
    n = B // BLK
    NB = min(2, n)

    def body(rows_hbm, emb_ref, num_ref, buf, sem):
        i = pl.program_id(0)

        @pl.when(i == 0)
        def _():
            pltpu.make_async_copy(
                rows_hbm.at[pl.ds(0, BLK)], buf.at[0], sem.at[0]).start()

        @pl.when(i + 1 < n)
        def _():
            pltpu.make_async_copy(
                rows_hbm.at[pl.ds((i + 1) * BLK, BLK)],
                buf.at[(i + 1) % 2], sem.at[(i + 1) % 2]).start()

        pltpu.make_async_copy(
            rows_hbm.at[pl.ds(i * BLK, BLK)],
            buf.at[i % 2], sem.at[i % 2]).wait()
        x = buf[i % 2]
        eye = (lax.broadcasted_iota(jnp.int32, (half, half), 0)
               == lax.broadcasted_iota(jnp.int32, (half, half), 1)
               ).astype(jnp.float32)
        dn = (((1,), (1,)), ((), ()))
        num_t = lax.dot_general(eye, x[:, :half], dn,
                                preferred_element_type=jnp.float32)
        den_t = lax.dot_general(eye, x[:, half:], dn,
                                preferred_element_type=jnp.float32)
        num_ref[...] = num_t
        emb_ref[...] = num_t / den_t

    return pl.pallas_call(
        body,
        grid=(n,),
        in_specs=[pl.BlockSpec(memory_space=pltpu.MemorySpace.HBM)],
        out_specs=[
            pl.BlockSpec((half, BLK), lambda i: (0, i)),
            pl.BlockSpec((half, BLK), lambda i: (0, i)),
        ],
        out_shape=(
            jax.ShapeDtypeStruct((half, B), jnp.float32),
            jax.ShapeDtypeStruct((half, B), jnp.float32),
        ),
        scratch_shapes=[
            pltpu.VMEM((NB, BLK, D), jnp.float32),
            pltpu.SemaphoreType.DMA((NB,)),
        ],
    )


def kernel(memory, nodes, memory_dim):
    V, D = memory.shape
    B = nodes.shape[0]
    half = D // 2
    rows = _build_gather(B, V, D)(memory, nodes.astype(jnp.int32))
    rows = pltpu.with_memory_space_constraint(rows, pltpu.MemorySpace.HBM)
    emb_t, num_t = _build_epilogue(B, D, half)(rows)
    return (emb_t.T, num_t.T)
